# Initial kernel scaffold; baseline (speedup 1.0000x reference)
import jax
import jax.numpy as jnp
from jax.experimental import pallas as pl


def kernel(x, Q, SEQ_HMM, SS_HMM, W1, b1, W2, b2):
    raise NotImplementedError("write your pallas kernel here")


# speedup vs baseline: 1.2645x; 1.2645x over previous
import jax
import jax.numpy as jnp
from jax.experimental import pallas as pl

SIZE = 512
NCH = 20
NCHP = 24
HPROF = 30
HPROFP = 32
CINP = NCHP + HPROFP
HID = 256
SSN = 3
KW = 5
PAD = KW // 2


def _shift_cols(v, s, iota):
    if s == 0:
        return v
    rolled = jnp.roll(v, -s, axis=1)
    valid = ((iota + s >= 0) & (iota + s < SIZE)).astype(v.dtype)
    return rolled * valid


def _dp_kernel(x_ref, seq_ref, ss_ref, w1_ref, b1_ref, w2_ref, b2_ref,
               out_ref):
    x = x_ref[0]
    m = (jnp.sum(x, axis=0, keepdims=True) > 0.0).astype(jnp.float32)
    hmm = jnp.concatenate([x, seq_ref[...] * m], axis=0)

    iota = jax.lax.broadcasted_iota(jnp.int32, (1, SIZE), 1)
    cols = jnp.concatenate(
        [_shift_cols(hmm, k - PAD, iota) for k in range(KW)], axis=0)
    h = jnp.dot(w1_ref[...], cols, preferred_element_type=jnp.float32)
    h = jnp.maximum(h + b1_ref[...], 0.0)

    logits = b2_ref[...] + jnp.zeros((SSN, SIZE), jnp.float32)
    for k in range(KW):
        logits = logits + jnp.dot(
            w2_ref[k], _shift_cols(h, k - PAD, iota),
            preferred_element_type=jnp.float32)

    mx = jnp.max(logits, axis=0, keepdims=True)
    e = jnp.exp(logits - mx)
    inv_e = jax.lax.rsqrt(jnp.sum(e * e, axis=0, keepdims=True))
    ss = ss_ref[0]
    ss_n = ss * jax.lax.rsqrt(jnp.sum(ss * ss, axis=0, keepdims=True))
    out_ref[0] = jnp.sum(e * ss_n, axis=0, keepdims=True) * inv_e * m


def kernel(x, Q, SEQ_HMM, SS_HMM, W1, b1, W2, b2):
    del Q
    B = x.shape[0]
    xp = jnp.pad(x, ((0, 0), (0, NCHP - NCH), (0, 0)))
    seqp = jnp.pad(SEQ_HMM, ((0, HPROFP - HPROF), (0, 0)))
    w1x = jnp.pad(W1[:, :NCH], ((0, 0), (0, NCHP - NCH), (0, 0)))
    w1s = jnp.pad(W1[:, NCH:], ((0, 0), (0, HPROFP - HPROF), (0, 0)))
    w1r = jnp.concatenate([w1x, w1s], axis=1)
    w1r = jnp.transpose(w1r, (0, 2, 1)).reshape(HID, KW * CINP)
    w2r = jnp.transpose(W2, (2, 0, 1))
    return pl.pallas_call(
        _dp_kernel,
        grid=(B,),
        in_specs=[
            pl.BlockSpec((1, NCHP, SIZE), lambda n: (n, 0, 0)),
            pl.BlockSpec((HPROFP, SIZE), lambda n: (0, 0)),
            pl.BlockSpec((1, SSN, SIZE), lambda n: (n, 0, 0)),
            pl.BlockSpec((HID, KW * CINP), lambda n: (0, 0)),
            pl.BlockSpec((HID, 1), lambda n: (0, 0)),
            pl.BlockSpec((KW, SSN, HID), lambda n: (0, 0, 0)),
            pl.BlockSpec((SSN, 1), lambda n: (0, 0)),
        ],
        out_specs=pl.BlockSpec((1, 1, SIZE), lambda n: (n, 0, 0)),
        out_shape=jax.ShapeDtypeStruct((B, 1, SIZE), jnp.float32),
    )(xp, seqp, SS_HMM, w1r, b1.reshape(HID, 1), w2r,
      b2.reshape(SSN, 1)).reshape(B, SIZE)


# speedup vs baseline: 1.4566x; 1.1519x over previous
import jax
import jax.numpy as jnp
from jax.experimental import pallas as pl

SIZE = 512
NCH = 20
NCHP = 24
HPROF = 30
HPROFP = 32
CINP = NCHP + HPROFP
HID = 256
SSN = 3
KW = 5
PAD = KW // 2


def _shift_cols(v, s, iota):
    if s == 0:
        return v
    rolled = jnp.roll(v, -s, axis=1)
    valid = ((iota + s >= 0) & (iota + s < SIZE)).astype(v.dtype)
    return rolled * valid


def _dp_kernel(x_ref, seq_ref, ss_ref, w1_ref, b1_ref, w2_ref, b2_ref,
               out_ref):
    x = x_ref[0]
    m = (jnp.sum(x, axis=0, keepdims=True) > 0.0).astype(jnp.float32)
    hmm = jnp.concatenate([x, seq_ref[...] * m], axis=0)

    iota = jax.lax.broadcasted_iota(jnp.int32, (1, SIZE), 1)
    cols = jnp.concatenate(
        [_shift_cols(hmm, k - PAD, iota) for k in range(KW)], axis=0)
    h = jnp.dot(w1_ref[...], cols, preferred_element_type=jnp.float32)
    h = jnp.maximum(h + b1_ref[...], 0.0)

    acc = jnp.dot(w2_ref[...], h, preferred_element_type=jnp.float32)
    lg = _shift_cols(acc[0:8], -PAD, iota)
    for k in range(1, KW):
        lg = lg + _shift_cols(acc[8 * k:8 * k + 8], k - PAD, iota)
    logits = lg[0:SSN] + b2_ref[...]

    mx = jnp.max(logits, axis=0, keepdims=True)
    e = jnp.exp(logits - mx)
    inv_e = jax.lax.rsqrt(jnp.sum(e * e, axis=0, keepdims=True))
    ss = ss_ref[0]
    ss_n = ss * jax.lax.rsqrt(jnp.sum(ss * ss, axis=0, keepdims=True))
    out_ref[0] = jnp.sum(e * ss_n, axis=0, keepdims=True) * inv_e * m


def kernel(x, Q, SEQ_HMM, SS_HMM, W1, b1, W2, b2):
    del Q
    B = x.shape[0]
    xp = jnp.pad(x, ((0, 0), (0, NCHP - NCH), (0, 0)))
    seqp = jnp.pad(SEQ_HMM, ((0, HPROFP - HPROF), (0, 0)))
    w1x = jnp.pad(W1[:, :NCH], ((0, 0), (0, NCHP - NCH), (0, 0)))
    w1s = jnp.pad(W1[:, NCH:], ((0, 0), (0, HPROFP - HPROF), (0, 0)))
    w1r = jnp.concatenate([w1x, w1s], axis=1)
    w1r = jnp.transpose(w1r, (0, 2, 1)).reshape(HID, KW * CINP)
    w2r = jnp.transpose(W2, (2, 0, 1))
    w2r = jnp.pad(w2r, ((0, 0), (0, 8 - SSN), (0, 0))).reshape(KW * 8, HID)
    return pl.pallas_call(
        _dp_kernel,
        grid=(B,),
        in_specs=[
            pl.BlockSpec((1, NCHP, SIZE), lambda n: (n, 0, 0)),
            pl.BlockSpec((HPROFP, SIZE), lambda n: (0, 0)),
            pl.BlockSpec((1, SSN, SIZE), lambda n: (n, 0, 0)),
            pl.BlockSpec((HID, KW * CINP), lambda n: (0, 0)),
            pl.BlockSpec((HID, 1), lambda n: (0, 0)),
            pl.BlockSpec((KW * 8, HID), lambda n: (0, 0)),
            pl.BlockSpec((SSN, 1), lambda n: (0, 0)),
        ],
        out_specs=pl.BlockSpec((1, 1, SIZE), lambda n: (n, 0, 0)),
        out_shape=jax.ShapeDtypeStruct((B, 1, SIZE), jnp.float32),
    )(xp, seqp, SS_HMM, w1r, b1.reshape(HID, 1), w2r,
      b2.reshape(SSN, 1)).reshape(B, SIZE)


# speedup vs baseline: 1.9499x; 1.3387x over previous
import jax
import jax.numpy as jnp
from jax.experimental import pallas as pl

SIZE = 512
NCH = 20
NCHP = 24
HPROF = 30
HPROFP = 32
CINP = NCHP + HPROFP
HID = 256
SSN = 3
KW = 5
PAD = KW // 2
SPB = 4


def _shift_cols(v, s, pos):
    if s == 0:
        return v
    rolled = jnp.roll(v, -s, axis=1)
    valid = ((pos + s >= 0) & (pos + s < SIZE)).astype(v.dtype)
    return rolled * valid


def _dp_kernel(x_ref, seq_ref, ss_ref, w1_ref, b1_ref, w2_ref, b2_ref,
               out_ref):
    x = x_ref[0]
    m = (jnp.sum(x, axis=0, keepdims=True) > 0.0).astype(jnp.float32)
    hmm = jnp.concatenate([x, seq_ref[...] * m], axis=0)

    lanes = x.shape[1]
    pos = jax.lax.broadcasted_iota(jnp.int32, (1, lanes), 1) & (SIZE - 1)
    cols = jnp.concatenate(
        [_shift_cols(hmm, k - PAD, pos) for k in range(KW)], axis=0)
    h = jnp.dot(w1_ref[...], cols, preferred_element_type=jnp.float32)
    h = jnp.maximum(h + b1_ref[...], 0.0)

    acc = jnp.dot(w2_ref[...], h, preferred_element_type=jnp.float32)
    lg = _shift_cols(acc[0:8], -PAD, pos)
    for k in range(1, KW):
        lg = lg + _shift_cols(acc[8 * k:8 * k + 8], k - PAD, pos)
    logits = lg[0:SSN] + b2_ref[...]

    mx = jnp.max(logits, axis=0, keepdims=True)
    e = jnp.exp(logits - mx)
    inv_e = jax.lax.rsqrt(jnp.sum(e * e, axis=0, keepdims=True))
    ss = ss_ref[0]
    ss_n = ss * jax.lax.rsqrt(jnp.sum(ss * ss, axis=0, keepdims=True))
    out_ref[0] = jnp.sum(e * ss_n, axis=0, keepdims=True) * inv_e * m


def kernel(x, Q, SEQ_HMM, SS_HMM, W1, b1, W2, b2):
    del Q
    B = x.shape[0]
    spb = SPB if B % SPB == 0 else 1
    G = B // spb
    L = spb * SIZE
    xp = jnp.pad(x, ((0, 0), (0, NCHP - NCH), (0, 0)))
    xp = xp.reshape(G, spb, NCHP, SIZE).transpose(0, 2, 1, 3).reshape(
        G, NCHP, L)
    seqp = jnp.pad(SEQ_HMM, ((0, HPROFP - HPROF), (0, 0)))
    seqp = jnp.tile(seqp, (1, spb))
    ssp = SS_HMM.reshape(G, spb, SSN, SIZE).transpose(0, 2, 1, 3).reshape(
        G, SSN, L)
    w1x = jnp.pad(W1[:, :NCH], ((0, 0), (0, NCHP - NCH), (0, 0)))
    w1s = jnp.pad(W1[:, NCH:], ((0, 0), (0, HPROFP - HPROF), (0, 0)))
    w1r = jnp.concatenate([w1x, w1s], axis=1)
    w1r = jnp.transpose(w1r, (0, 2, 1)).reshape(HID, KW * CINP)
    w2r = jnp.transpose(W2, (2, 0, 1))
    w2r = jnp.pad(w2r, ((0, 0), (0, 8 - SSN), (0, 0))).reshape(KW * 8, HID)
    out = pl.pallas_call(
        _dp_kernel,
        grid=(G,),
        in_specs=[
            pl.BlockSpec((1, NCHP, L), lambda n: (n, 0, 0)),
            pl.BlockSpec((HPROFP, L), lambda n: (0, 0)),
            pl.BlockSpec((1, SSN, L), lambda n: (n, 0, 0)),
            pl.BlockSpec((HID, KW * CINP), lambda n: (0, 0)),
            pl.BlockSpec((HID, 1), lambda n: (0, 0)),
            pl.BlockSpec((KW * 8, HID), lambda n: (0, 0)),
            pl.BlockSpec((SSN, 1), lambda n: (0, 0)),
        ],
        out_specs=pl.BlockSpec((1, 1, L), lambda n: (n, 0, 0)),
        out_shape=jax.ShapeDtypeStruct((G, 1, L), jnp.float32),
    )(xp, seqp, ssp, w1r, b1.reshape(HID, 1), w2r, b2.reshape(SSN, 1))
    return out.reshape(B, SIZE)


# speedup vs baseline: 2.0067x; 1.0291x over previous
import jax
import jax.numpy as jnp
from jax.experimental import pallas as pl

SIZE = 512
NCH = 20
NCHP = 24
HPROF = 30
HPROFP = 32
CINP = NCHP + HPROFP
HID = 256
SSN = 3
KW = 5
PAD = KW // 2
SPB = 8


def _shift_cols(v, s, pos):
    if s == 0:
        return v
    rolled = jnp.roll(v, -s, axis=1)
    valid = ((pos + s >= 0) & (pos + s < SIZE)).astype(v.dtype)
    return rolled * valid


def _dp_kernel(x_ref, seq_ref, ss_ref, w1_ref, b1_ref, w2_ref, b2_ref,
               out_ref):
    x = x_ref[0]
    m = (jnp.sum(x, axis=0, keepdims=True) > 0.0).astype(jnp.float32)
    hmm = jnp.concatenate([x, seq_ref[...] * m], axis=0)

    lanes = x.shape[1]
    pos = jax.lax.broadcasted_iota(jnp.int32, (1, lanes), 1) & (SIZE - 1)
    cols = jnp.concatenate(
        [_shift_cols(hmm, k - PAD, pos) for k in range(KW)], axis=0)
    h = jnp.dot(w1_ref[...], cols, preferred_element_type=jnp.float32)
    h = jnp.maximum(h + b1_ref[...], 0.0)

    acc = jnp.dot(w2_ref[...], h, preferred_element_type=jnp.float32)
    lg = _shift_cols(acc[0:8], -PAD, pos)
    for k in range(1, KW):
        lg = lg + _shift_cols(acc[8 * k:8 * k + 8], k - PAD, pos)
    logits = lg[0:SSN] + b2_ref[...]

    mx = jnp.max(logits, axis=0, keepdims=True)
    e = jnp.exp(logits - mx)
    inv_e = jax.lax.rsqrt(jnp.sum(e * e, axis=0, keepdims=True))
    ss = ss_ref[0]
    ss_n = ss * jax.lax.rsqrt(jnp.sum(ss * ss, axis=0, keepdims=True))
    out_ref[0] = jnp.sum(e * ss_n, axis=0, keepdims=True) * inv_e * m


def kernel(x, Q, SEQ_HMM, SS_HMM, W1, b1, W2, b2):
    del Q
    B = x.shape[0]
    spb = SPB if B % SPB == 0 else 1
    G = B // spb
    L = spb * SIZE
    xp = jnp.pad(x, ((0, 0), (0, NCHP - NCH), (0, 0)))
    xp = xp.reshape(G, spb, NCHP, SIZE).transpose(0, 2, 1, 3).reshape(
        G, NCHP, L)
    seqp = jnp.pad(SEQ_HMM, ((0, HPROFP - HPROF), (0, 0)))
    seqp = jnp.tile(seqp, (1, spb))
    ssp = SS_HMM.reshape(G, spb, SSN, SIZE).transpose(0, 2, 1, 3).reshape(
        G, SSN, L)
    w1x = jnp.pad(W1[:, :NCH], ((0, 0), (0, NCHP - NCH), (0, 0)))
    w1s = jnp.pad(W1[:, NCH:], ((0, 0), (0, HPROFP - HPROF), (0, 0)))
    w1r = jnp.concatenate([w1x, w1s], axis=1)
    w1r = jnp.transpose(w1r, (0, 2, 1)).reshape(HID, KW * CINP)
    w2r = jnp.transpose(W2, (2, 0, 1))
    w2r = jnp.pad(w2r, ((0, 0), (0, 8 - SSN), (0, 0))).reshape(KW * 8, HID)
    out = pl.pallas_call(
        _dp_kernel,
        grid=(G,),
        in_specs=[
            pl.BlockSpec((1, NCHP, L), lambda n: (n, 0, 0)),
            pl.BlockSpec((HPROFP, L), lambda n: (0, 0)),
            pl.BlockSpec((1, SSN, L), lambda n: (n, 0, 0)),
            pl.BlockSpec((HID, KW * CINP), lambda n: (0, 0)),
            pl.BlockSpec((HID, 1), lambda n: (0, 0)),
            pl.BlockSpec((KW * 8, HID), lambda n: (0, 0)),
            pl.BlockSpec((SSN, 1), lambda n: (0, 0)),
        ],
        out_specs=pl.BlockSpec((1, 1, L), lambda n: (n, 0, 0)),
        out_shape=jax.ShapeDtypeStruct((G, 1, L), jnp.float32),
    )(xp, seqp, ssp, w1r, b1.reshape(HID, 1), w2r, b2.reshape(SSN, 1))
    return out.reshape(B, SIZE)
